# baseline (speedup 1.0000x reference)
import functools

import jax
import jax.numpy as jnp
from jax import lax
from jax.experimental import pallas as pl
from jax.experimental.pallas import tpu as pltpu
from jax.experimental.pallas import tpu_sc as plsc

N = 10000
E = 320000
D = 128

NC = 2
NS = 16
NW = NC * NS

EB = 128
N_PAD = 10240
K = 2
NG = 40
NB = NG * K
E_PAD = NW * NB * EB
E_HBM_GROUPS = NG + 1
RPT = N_PAD // NS

_MESH = plsc.VectorSubcoreMesh(core_axis_name="c", subcore_axis_name="s")


def _deg_body(dst_hbm, z16_hbm, out0, out1, dst_v, ones, acc, sem):
    c = lax.axis_index("c")
    s = lax.axis_index("s")
    wid = s * NC + c
    r0 = s * RPT

    pltpu.async_copy(z16_hbm.at[pl.ds(r0, RPT)], acc.at[pl.ds(r0, RPT)], sem).wait()
    pltpu.sync_copy(dst_hbm.at[wid], dst_v)

    def fill(i, carry):
        ones[i, :] = jnp.ones((16,), jnp.float32)
        return carry

    lax.fori_loop(0, EB, fill, 0)
    plsc.subcore_barrier()

    def body(i, carry):
        pltpu.sync_copy(ones, acc.at[dst_v.at[i]], add=True)
        return carry

    lax.fori_loop(0, NB, body, 0)
    plsc.subcore_barrier()

    @pl.when(c == 0)
    def _():
        pltpu.async_copy(acc.at[pl.ds(r0, RPT)], out0.at[pl.ds(r0, RPT)], sem).wait()

    @pl.when(c == 1)
    def _():
        pltpu.async_copy(acc.at[pl.ds(r0, RPT)], out1.at[pl.ds(r0, RPT)], sem).wait()


_deg_call = pl.kernel(
    _deg_body,
    out_type=(
        jax.ShapeDtypeStruct((N_PAD, 16), jnp.float32),
        jax.ShapeDtypeStruct((N_PAD, 16), jnp.float32),
    ),
    mesh=_MESH,
    scratch_types=[
        pltpu.VMEM((NB, EB), jnp.int32),
        pltpu.VMEM((EB, 16), jnp.float32),
        pltpu.VMEM_SHARED((N_PAD, 16), jnp.float32),
        pltpu.SemaphoreType.DMA,
    ],
)


def _gcn_body(y_hbm, z_hbm, src_hbm, dst_hbm, out0, out1,
              si0, si1, di0, di1, rb0, rb1,
              acc, g0, g1, s0, s1, x0, x1, x2, x3, isem):
    rows = [rb0, rb1]
    si = [si0, si1]
    di = [di0, di1]
    gsem = [g0, g1]
    ssem = [s0, s1]
    sxs = [x0, x1]
    dxs = [x2, x3]
    c = lax.axis_index("c")
    s = lax.axis_index("s")
    wid = s * NC + c
    r0 = s * RPT

    pltpu.async_copy(src_hbm.at[wid, 0], si[0], sxs[0])
    pltpu.async_copy(dst_hbm.at[wid, 0], di[0], dxs[0])

    @pl.when(c == 0)
    def _():
        pltpu.async_copy(y_hbm.at[pl.ds(r0, RPT)], acc.at[pl.ds(r0, RPT)], isem).wait()

    @pl.when(c == 1)
    def _():
        pltpu.async_copy(z_hbm.at[pl.ds(r0, RPT)], acc.at[pl.ds(r0, RPT)], isem).wait()

    plsc.subcore_barrier()

    def pair(p, carry):
        for q in range(2):
            g = 2 * p + q
            pltpu.async_copy(src_hbm.at[wid, g + 1], si[1 - q], sxs[1 - q])
            pltpu.async_copy(dst_hbm.at[wid, g + 1], di[1 - q], dxs[1 - q])
            pltpu.make_async_copy(src_hbm.at[wid, 0], si[q], sxs[q]).wait()
            pltpu.make_async_copy(dst_hbm.at[wid, 0], di[q], dxs[q]).wait()

            descs = []
            for b in range(K):
                descs.append(pltpu.async_copy(y_hbm.at[si[q].at[b]], rows[b], gsem[b]))
            for b in range(K):
                descs[b].wait()
                pltpu.sync_copy(rows[b], acc.at[di[q].at[b]], add=True)
        return carry

    lax.fori_loop(0, NG // 2, pair, 0)
    pltpu.make_async_copy(src_hbm.at[wid, 0], si[0], sxs[0]).wait()
    pltpu.make_async_copy(dst_hbm.at[wid, 0], di[0], dxs[0]).wait()
    plsc.subcore_barrier()

    @pl.when(c == 0)
    def _():
        pltpu.async_copy(acc.at[pl.ds(r0, RPT)], out0.at[pl.ds(r0, RPT)], isem).wait()

    @pl.when(c == 1)
    def _():
        pltpu.async_copy(acc.at[pl.ds(r0, RPT)], out1.at[pl.ds(r0, RPT)], isem).wait()


_gcn_call = pl.kernel(
    _gcn_body,
    out_type=(
        jax.ShapeDtypeStruct((N_PAD, D), jnp.float32),
        jax.ShapeDtypeStruct((N_PAD, D), jnp.float32),
    ),
    mesh=_MESH,
    scratch_types=[pltpu.VMEM((K, EB), jnp.int32)] * 4
    + [pltpu.VMEM((EB, D), jnp.float32)] * K
    + [pltpu.VMEM_SHARED((N_PAD, D), jnp.float32)]
    + [pltpu.SemaphoreType.DMA] * 9,
)


BN = 640
_GRID = (N_PAD // BN,)
_row_spec = pl.BlockSpec((BN, D), lambda i: (i, 0))
_deg_spec = pl.BlockSpec((BN, 16), lambda i: (i, 0))
_w_spec = pl.BlockSpec((D, D), lambda i: (0, 0))
_b_spec = pl.BlockSpec((1, D), lambda i: (0, 0))
_f32 = functools.partial(jax.ShapeDtypeStruct, dtype=jnp.float32)


def _dinv(d0_ref, d1_ref):
    return lax.rsqrt(d0_ref[:, :1] + d1_ref[:, :1] + 1.0)


def _mm_body(x_ref, w_ref, o_ref):
    o_ref[:, :] = jnp.dot(x_ref[:, :], w_ref[:, :], preferred_element_type=jnp.float32)


_mm_call = pl.pallas_call(
    _mm_body,
    grid=_GRID,
    in_specs=[_row_spec, _w_spec],
    out_specs=_row_spec,
    out_shape=_f32((N_PAD, D)),
)


def _scale_body(xw_ref, d0_ref, d1_ref, o_ref):
    o_ref[:, :] = xw_ref[:, :] * _dinv(d0_ref, d1_ref)


_scale_call = pl.pallas_call(
    _scale_body,
    grid=_GRID,
    in_specs=[_row_spec, _deg_spec, _deg_spec],
    out_specs=_row_spec,
    out_shape=_f32((N_PAD, D)),
)


def _mid_body(a0_ref, a1_ref, d0_ref, d1_ref, b1_ref, w2_ref, o_ref):
    dinv = _dinv(d0_ref, d1_ref)
    h = jnp.maximum((a0_ref[:, :] + a1_ref[:, :]) * dinv + b1_ref[:, :], 0.0)
    o_ref[:, :] = jnp.dot(h, w2_ref[:, :], preferred_element_type=jnp.float32) * dinv


_mid_call = pl.pallas_call(
    _mid_body,
    grid=_GRID,
    in_specs=[_row_spec, _row_spec, _deg_spec, _deg_spec, _b_spec, _w_spec],
    out_specs=_row_spec,
    out_shape=_f32((N_PAD, D)),
)


def _fin_body(a0_ref, a1_ref, d0_ref, d1_ref, b2_ref, o_ref):
    o_ref[:, :] = (a0_ref[:, :] + a1_ref[:, :]) * _dinv(d0_ref, d1_ref) + b2_ref[:, :]


_fin_call = pl.pallas_call(
    _fin_body,
    grid=_GRID,
    in_specs=[_row_spec, _row_spec, _deg_spec, _deg_spec, _b_spec],
    out_specs=_row_spec,
    out_shape=_f32((N_PAD, D)),
)


def kernel(x, edge_index, W1, b1, W2, b2):
    src = edge_index[0].astype(jnp.int32)
    dst = edge_index[1].astype(jnp.int32)
    npad_e = E_PAD - E
    pad_src = jnp.full((npad_e,), N, jnp.int32)
    pad_dst = N + (jnp.arange(npad_e, dtype=jnp.int32) % (N_PAD - N))
    src_b = jnp.concatenate([src, pad_src]).reshape(NW, NG, K, EB)
    dst_b = jnp.concatenate([dst, pad_dst]).reshape(NW, NG, K, EB)
    extra = jnp.full((NW, 1, K, EB), N, jnp.int32)
    src4 = jnp.concatenate([src_b, extra], axis=1)
    dst4 = jnp.concatenate([dst_b, extra], axis=1)
    dst3 = dst_b.reshape(NW, NB, EB)

    x_p = jnp.zeros((N_PAD, D), jnp.float32).at[:N].set(x)
    z2 = jnp.zeros((N_PAD, D), jnp.float32)
    z16 = jnp.zeros((N_PAD, 16), jnp.float32)

    d0, d1 = _deg_call(dst3, z16)
    xw1 = _mm_call(x_p, W1)
    y1 = _scale_call(xw1, d0, d1)
    a0, a1 = _gcn_call(y1, z2, src4, dst4)
    y2 = _mid_call(a0, a1, d0, d1, b1.reshape(1, D), W2)
    a0b, a1b = _gcn_call(y2, z2, src4, dst4)
    out = _fin_call(a0b, a1b, d0, d1, b2.reshape(1, D))
    return out[:N]


# speedup vs baseline: 1.3422x; 1.3422x over previous
import functools

import jax
import jax.numpy as jnp
from jax import lax
from jax.experimental import pallas as pl
from jax.experimental.pallas import tpu as pltpu
from jax.experimental.pallas import tpu_sc as plsc

N = 10000
E = 320000
D = 128

NC = 2
NS = 16
NW = NC * NS

EB = 128
N_PAD = 10240
K = 2
NG0 = 64
NG1 = 16
GP0 = NG0 + 1
GP1 = NG1 + 1
GE = K * EB
E_PAD = NS * (NG0 + NG1) * GE
NB_D = 79
E_PAD_D = NW * NB_D * EB
RPT = N_PAD // NS

_MESH = plsc.VectorSubcoreMesh(core_axis_name="c", subcore_axis_name="s")


def _deg_body(dst_hbm, z16_hbm, out, dst_v, ones, acc, sem):
    c = lax.axis_index("c")
    s = lax.axis_index("s")
    wid = s * NC + c
    r0 = s * RPT

    pltpu.async_copy(z16_hbm.at[pl.ds(r0, RPT)], acc.at[pl.ds(r0, RPT)], sem).wait()
    pltpu.sync_copy(dst_hbm.at[wid], dst_v)

    def fill(i, carry):
        ones[i, :] = jnp.ones((16,), jnp.float32)
        return carry

    lax.fori_loop(0, EB, fill, 0)
    plsc.subcore_barrier()

    def body(i, carry):
        pltpu.sync_copy(ones, acc.at[dst_v.at[i]], add=True)
        return carry

    lax.fori_loop(0, NB_D, body, 0)
    plsc.subcore_barrier()
    pltpu.async_copy(acc.at[pl.ds(r0, RPT)], out.at[c, pl.ds(r0, RPT)], sem).wait()


_deg_call = pl.kernel(
    _deg_body,
    out_type=jax.ShapeDtypeStruct((NC, N_PAD, 16), jnp.float32),
    mesh=_MESH,
    scratch_types=[
        pltpu.VMEM((NB_D, EB), jnp.int32),
        pltpu.VMEM((EB, 16), jnp.float32),
        pltpu.VMEM_SHARED((N_PAD, 16), jnp.float32),
        pltpu.SemaphoreType.DMA,
    ],
)


def _gcn_body(y_hbm, z_hbm, src_hbm, dst_hbm, out,
              si0, si1, di0, di1, rb0, rb1,
              acc, g0, g1, s0, s1, x0, x1, x2, x3, isem):
    rows = [rb0, rb1]
    si = [si0, si1]
    di = [di0, di1]
    gsem = [g0, g1]
    ssem = [s0, s1]
    sxs = [x0, x1]
    dxs = [x2, x3]
    c = lax.axis_index("c")
    s = lax.axis_index("s")
    r0 = s * RPT
    tbase = (1 - c) * (s * GP0) + c * (NS * GP0 + s * GP1)
    ng = (1 - c) * NG0 + c * NG1

    pltpu.async_copy(src_hbm.at[tbase], si[0], sxs[0])
    pltpu.async_copy(dst_hbm.at[tbase], di[0], dxs[0])

    pltpu.async_copy(z_hbm.at[pl.ds(r0, RPT)], acc.at[pl.ds(r0, RPT)], isem).wait()
    plsc.subcore_barrier()

    def pair(p, carry):
        for q in range(2):
            g = 2 * p + q

            @pl.when(g < ng)
            def _(g=g, q=q):
                pltpu.async_copy(src_hbm.at[tbase + g + 1], si[1 - q], sxs[1 - q])
                pltpu.async_copy(dst_hbm.at[tbase + g + 1], di[1 - q], dxs[1 - q])
                pltpu.make_async_copy(src_hbm.at[tbase], si[q], sxs[q]).wait()
                pltpu.make_async_copy(dst_hbm.at[tbase], di[q], dxs[q]).wait()

                descs = []
                for b in range(K):
                    descs.append(pltpu.async_copy(y_hbm.at[si[q].at[b]], rows[b], gsem[b]))
                for b in range(K):
                    descs[b].wait()
                    pltpu.sync_copy(rows[b], acc.at[di[q].at[b]], add=True)
        return carry

    lax.fori_loop(0, NG0 // 2, pair, 0)
    pltpu.make_async_copy(src_hbm.at[tbase], si[0], sxs[0]).wait()
    pltpu.make_async_copy(dst_hbm.at[tbase], di[0], dxs[0]).wait()
    plsc.subcore_barrier()
    pltpu.async_copy(acc.at[pl.ds(r0, RPT)], out.at[c, pl.ds(r0, RPT)], isem).wait()


_gcn_call = pl.kernel(
    _gcn_body,
    out_type=jax.ShapeDtypeStruct((NC, N_PAD, D), jnp.float32),
    mesh=_MESH,
    scratch_types=[pltpu.VMEM((K, EB), jnp.int32)] * 4
    + [pltpu.VMEM((EB, D), jnp.float32)] * K
    + [pltpu.VMEM_SHARED((N_PAD, D), jnp.float32)]
    + [pltpu.SemaphoreType.DMA] * 9,
)


BN = 640
_GRID = (N_PAD // BN,)
_row_spec = pl.BlockSpec((BN, D), lambda i: (i, 0))
_acc_spec = pl.BlockSpec((NC, BN, D), lambda i: (0, i, 0))
_deg_spec = pl.BlockSpec((NC, BN, 16), lambda i: (0, i, 0))
_w_spec = pl.BlockSpec((D, D), lambda i: (0, 0))
_b_spec = pl.BlockSpec((1, D), lambda i: (0, 0))
_f32 = functools.partial(jax.ShapeDtypeStruct, dtype=jnp.float32)


def _dinv(dp_ref):
    return lax.rsqrt(dp_ref[0, :, :1] + dp_ref[1, :, :1] + 1.0)


def _mm_body(x_ref, w_ref, o_ref):
    o_ref[:, :] = jnp.dot(x_ref[:, :], w_ref[:, :], preferred_element_type=jnp.float32)


_mm_call = pl.pallas_call(
    _mm_body,
    grid=_GRID,
    in_specs=[_row_spec, _w_spec],
    out_specs=_row_spec,
    out_shape=_f32((N_PAD, D)),
)


def _scale_body(xw_ref, dp_ref, o_ref):
    o_ref[:, :] = xw_ref[:, :] * _dinv(dp_ref)


_scale_call = pl.pallas_call(
    _scale_body,
    grid=_GRID,
    in_specs=[_row_spec, _deg_spec],
    out_specs=_row_spec,
    out_shape=_f32((N_PAD, D)),
)


def _mid_body(a_ref, y_ref, dp_ref, b1_ref, w2_ref, o_ref):
    dinv = _dinv(dp_ref)
    agg = a_ref[0, :, :] + a_ref[1, :, :] + y_ref[:, :]
    h = jnp.maximum(agg * dinv + b1_ref[:, :], 0.0)
    o_ref[:, :] = jnp.dot(h, w2_ref[:, :], preferred_element_type=jnp.float32) * dinv


_mid_call = pl.pallas_call(
    _mid_body,
    grid=_GRID,
    in_specs=[_acc_spec, _row_spec, _deg_spec, _b_spec, _w_spec],
    out_specs=_row_spec,
    out_shape=_f32((N_PAD, D)),
)


def _fin_body(a_ref, y_ref, dp_ref, b2_ref, o_ref):
    agg = a_ref[0, :, :] + a_ref[1, :, :] + y_ref[:, :]
    o_ref[:, :] = agg * _dinv(dp_ref) + b2_ref[:, :]


_fin_call = pl.pallas_call(
    _fin_body,
    grid=_GRID,
    in_specs=[_acc_spec, _row_spec, _deg_spec, _b_spec],
    out_specs=_row_spec,
    out_shape=_f32((N_PAD, D)),
)


def kernel(x, edge_index, W1, b1, W2, b2):
    src = edge_index[0].astype(jnp.int32)
    dst = edge_index[1].astype(jnp.int32)
    npad_e = E_PAD - E
    pad_src = jnp.full((npad_e,), N, jnp.int32)
    pad_dst = N + (jnp.arange(npad_e, dtype=jnp.int32) % (N_PAD - N))
    sfull = jnp.concatenate([src, pad_src]).reshape(-1, GE)
    dfull = jnp.concatenate([dst, pad_dst]).reshape(-1, GE)

    def lay(a):
        g0 = a[:NS * NG0].reshape(NS, NG0, GE)
        g1 = a[NS * NG0:].reshape(NS, NG1, GE)
        ex = jnp.full((NS, 1, GE), N, jnp.int32)
        return jnp.concatenate([
            jnp.concatenate([g0, ex], 1).reshape(-1, GE),
            jnp.concatenate([g1, ex], 1).reshape(-1, GE),
        ]).reshape(-1, K, EB)

    src4 = lay(sfull)
    dst4 = lay(dfull)
    npad_d = E_PAD_D - E
    dst3 = jnp.concatenate(
        [dst, N + (jnp.arange(npad_d, dtype=jnp.int32) % (N_PAD - N))]
    ).reshape(NW, NB_D, EB)

    x_p = jnp.zeros((N_PAD, D), jnp.float32).at[:N].set(x)
    z2 = jnp.zeros((N_PAD, D), jnp.float32)
    z16 = jnp.zeros((N_PAD, 16), jnp.float32)

    dp = _deg_call(dst3, z16)
    xw1 = _mm_call(x_p, W1)
    y1 = _scale_call(xw1, dp)
    a1l = _gcn_call(y1, z2, src4, dst4)
    y2 = _mid_call(a1l, y1, dp, b1.reshape(1, D), W2)
    a2l = _gcn_call(y2, z2, src4, dst4)
    out = _fin_call(a2l, y2, dp, b2.reshape(1, D))
    return out[:N]
